# Initial kernel scaffold; baseline (speedup 1.0000x reference)
import jax
import jax.numpy as jnp
from jax.experimental import pallas as pl


def kernel(input_sequence, A_diag_raw, B_real, B_img, C_real, C_img, D, steps_raw):
    raise NotImplementedError("write your pallas kernel here")


# speedup vs baseline: 17.6439x; 17.6439x over previous
import jax
import jax.numpy as jnp
from jax.experimental import pallas as pl
from jax.experimental.pallas import tpu as pltpu

_T = 512


def _linoss_body(x_ref, btr_ref, bti_ref, ctr_ref, cti_ref, d_ref, ad_ref,
                 st_ref, o_ref, carry_ref):
    i = pl.program_id(0)
    T = x_ref.shape[0]
    P = ad_ref.shape[1]

    a = jnp.maximum(ad_ref[...], 0.0)
    s = jax.nn.sigmoid(st_ref[...])
    mA = jnp.ones_like(s)
    mB = -s * a
    mC = s
    mD = 1.0 - s * s * a

    x = x_ref[...]
    bur = jnp.dot(x, btr_ref[...], preferred_element_type=jnp.float32)
    bui = jnp.dot(x, bti_ref[...], preferred_element_type=jnp.float32)
    f_r = bur * s
    f_i = bui * s

    @pl.when(i == 0)
    def _():
        carry_ref[...] = jnp.zeros_like(carry_ref)

    c = carry_ref[...]
    c1r, c1i, c2r, c2i = c[0:1], c[1:2], c[2:3], c[3:4]
    d1r = mA * c1r + mB * c2r
    d1i = mA * c1i + mB * c2i
    d2r = mC * c1r + mD * c2r
    d2i = mC * c1i + mD * c2i

    rowmask = (jax.lax.broadcasted_iota(jnp.int32, (T, 1), 0) == 0
               ).astype(jnp.float32)
    b1r = f_r + rowmask * d1r
    b1i = f_i + rowmask * d1i
    b2r = f_r + rowmask * d2r
    b2i = f_i + rowmask * d2i

    nA, nB, nC, nD = mA, mB, mC, mD
    d = 1
    while d < T:
        z = jnp.zeros((d, P), jnp.float32)
        s1r = jnp.concatenate([z, b1r[:T - d]], axis=0)
        s1i = jnp.concatenate([z, b1i[:T - d]], axis=0)
        s2r = jnp.concatenate([z, b2r[:T - d]], axis=0)
        s2i = jnp.concatenate([z, b2i[:T - d]], axis=0)
        b1r = b1r + nA * s1r + nB * s2r
        b1i = b1i + nA * s1i + nB * s2i
        b2r = b2r + nC * s1r + nD * s2r
        b2i = b2i + nC * s1i + nD * s2i
        if d * 2 < T:
            tr = nA + nD
            nA, nB, nC, nD = nA * nA + nB * nC, nB * tr, nC * tr, nD * nD + nB * nC
        d *= 2

    carry_ref[0:1] = b1r[T - 1:T]
    carry_ref[1:2] = b1i[T - 1:T]
    carry_ref[2:3] = b2r[T - 1:T]
    carry_ref[3:4] = b2i[T - 1:T]

    o = (jnp.dot(b2r, ctr_ref[...], preferred_element_type=jnp.float32)
         - jnp.dot(b2i, cti_ref[...], preferred_element_type=jnp.float32)
         + x * d_ref[...])
    o_ref[...] = o


def kernel(input_sequence, A_diag_raw, B_real, B_img, C_real, C_img, D,
           steps_raw):
    L, H = input_sequence.shape
    P = A_diag_raw.shape[0]
    n_chunks = L // _T

    return pl.pallas_call(
        _linoss_body,
        out_shape=jax.ShapeDtypeStruct((L, H), jnp.float32),
        grid=(n_chunks,),
        in_specs=[
            pl.BlockSpec((_T, H), lambda i: (i, 0)),
            pl.BlockSpec((H, P), lambda i: (0, 0)),
            pl.BlockSpec((H, P), lambda i: (0, 0)),
            pl.BlockSpec((P, H), lambda i: (0, 0)),
            pl.BlockSpec((P, H), lambda i: (0, 0)),
            pl.BlockSpec((1, H), lambda i: (0, 0)),
            pl.BlockSpec((1, P), lambda i: (0, 0)),
            pl.BlockSpec((1, P), lambda i: (0, 0)),
        ],
        out_specs=pl.BlockSpec((_T, H), lambda i: (i, 0)),
        scratch_shapes=[pltpu.VMEM((8, P), jnp.float32)],
        compiler_params=pltpu.CompilerParams(
            dimension_semantics=("arbitrary",),
        ),
        name="linoss_scan",
    )(
        input_sequence,
        B_real.T, B_img.T,
        C_real.T, C_img.T,
        D.reshape(1, H),
        A_diag_raw.reshape(1, P),
        steps_raw.reshape(1, P),
    )
